# baseline (speedup 1.0000x reference)
import functools
import math

import jax
import jax.numpy as jnp
from jax import lax
from jax.experimental import pallas as pl
from jax.experimental.pallas import tpu as pltpu
from jax.experimental.pallas import tpu_sc as plsc

N = 10000
IN_DIM = 128
H = 64
NCLS = 6
NG = 16
EPS = 1e-5

SC_CORES = 2
SC_TILES = 16
NW = SC_CORES * SC_TILES
CL = 128
E = 320000
NBUF = 4
NCH = -(-(-(-(E // NW) // CL)) // (2 * NBUF)) * (2 * NBUF)
NT = NCH // NBUF
EP = NW * NCH * CL
NP = 10240
RP = NP // SC_TILES
DEGW = 16

BR = 1024
GRID = NP // BR

_INV_BN = 1.0 / math.sqrt(1.0 + EPS)


def _sc_mesh():
    return plsc.VectorSubcoreMesh(
        core_axis_name="c", subcore_axis_name="s",
        num_cores=SC_CORES, num_subcores=SC_TILES)


def _deg_call(dstp, ones_h, zeros_h):
    @functools.partial(
        pl.kernel,
        out_type=jax.ShapeDtypeStruct((SC_CORES, NP, DEGW), jnp.float32),
        mesh=_sc_mesh(),
        scratch_types=[
            pltpu.VMEM((NCH, CL), jnp.int32),
            pltpu.VMEM((CL, DEGW), jnp.float32),
            pltpu.VMEM_SHARED((NP, DEGW), jnp.float32),
        ],
        compiler_params=pltpu.CompilerParams(use_tc_tiling_on_sc=False),
    )
    def k(dst_hbm, ones_hbm, zeros_hbm, out_hbm, didx, ones_v, acc):
        c = lax.axis_index("c")
        s = lax.axis_index("s")
        wid = c * SC_TILES + s
        for b in range(NBUF):
            pltpu.sync_copy(dst_hbm.at[b, wid, pl.ds(0, NT)],
                            didx.at[pl.ds(b * NT, NT)])
        pltpu.sync_copy(ones_hbm, ones_v)
        pltpu.sync_copy(zeros_hbm, acc.at[pl.ds(s * RP, RP)])
        plsc.subcore_barrier()

        def step(j, carry):
            pltpu.sync_copy(ones_v, acc.at[didx.at[j]], add=True)
            return carry
        lax.fori_loop(0, NCH, step, 0)
        plsc.subcore_barrier()
        pltpu.sync_copy(acc.at[pl.ds(s * RP, RP)],
                        out_hbm.at[c, pl.ds(s * RP, RP)])

    return k(dstp, ones_h, zeros_h)


def _scatter_call(hp, srcp, dstp, zeros_h):
    @functools.partial(
        pl.kernel,
        out_type=jax.ShapeDtypeStruct((SC_CORES, NP, H), jnp.float32),
        mesh=_sc_mesh(),
        scratch_types=(
            [pltpu.VMEM((NT + 1, CL), jnp.int32) for _ in range(2 * NBUF)]
            + [pltpu.VMEM((CL, H), jnp.float32) for _ in range(2 * NBUF)]
            + [pltpu.VMEM_SHARED((NP, H), jnp.float32),
               pltpu.SemaphoreType.DMA, pltpu.SemaphoreType.DMA]
        ),
        compiler_params=pltpu.CompilerParams(use_tc_tiling_on_sc=False),
    )
    def k(hp_hbm, src_hbm, dst_hbm, zeros_hbm, out_hbm, *rest):
        sidx = rest[:NBUF]
        didx = rest[NBUF:2 * NBUF]
        rows_a = rest[2 * NBUF:3 * NBUF]
        rows_b = rest[3 * NBUF:4 * NBUF]
        acc = rest[4 * NBUF]
        sem_a = rest[4 * NBUF + 1]
        sem_b = rest[4 * NBUF + 2]
        c = lax.axis_index("c")
        s = lax.axis_index("s")
        wid = c * SC_TILES + s
        for b in range(NBUF):
            pltpu.sync_copy(src_hbm.at[b, wid], sidx[b])
            pltpu.sync_copy(dst_hbm.at[b, wid], didx[b])
        pltpu.sync_copy(zeros_hbm, acc.at[pl.ds(s * RP, RP)])
        plsc.subcore_barrier()

        def drain(rows, sem):
            for b in range(NBUF):
                pltpu.make_async_copy(
                    hp_hbm.at[sidx[b].at[0]], rows[b], sem).wait()

        for b in range(NBUF):
            pltpu.async_copy(hp_hbm.at[sidx[b].at[0]], rows_a[b], sem_a)

        def outer(u, j):
            drain(rows_a, sem_a)
            for b in range(NBUF):
                pltpu.async_copy(
                    hp_hbm.at[sidx[b].at[j + 1]], rows_b[b], sem_b)
            for b in range(NBUF):
                pltpu.sync_copy(rows_a[b], acc.at[didx[b].at[j]], add=True)
            drain(rows_b, sem_b)
            for b in range(NBUF):
                pltpu.async_copy(
                    hp_hbm.at[sidx[b].at[j + 2]], rows_a[b], sem_a)
            for b in range(NBUF):
                pltpu.sync_copy(rows_b[b], acc.at[didx[b].at[j + 1]],
                                add=True)
            return j + 2
        lax.fori_loop(0, NT // 2, outer, jnp.int32(0))
        drain(rows_a, sem_a)
        plsc.subcore_barrier()
        pltpu.sync_copy(acc.at[pl.ds(s * RP, RP)],
                        out_hbm.at[c, pl.ds(s * RP, RP)])

    return k(hp, srcp, dstp, zeros_h)


def _lead_call(xp, deg0, deg1, W0, b0, Wc0):
    def body(x_ref, d0_ref, d1_ref, w0_ref, b0_ref, wc_ref, hp_ref, dinv_ref):
        dinv = lax.rsqrt(d0_ref[...] + d1_ref[...] + 1.0)
        h = jnp.dot(x_ref[...], w0_ref[...], preferred_element_type=jnp.float32)
        h = jnp.maximum(h + b0_ref[...], 0.0)
        hp = jnp.dot(h, wc_ref[...], preferred_element_type=jnp.float32)
        hp_ref[...] = hp * dinv
        dinv_ref[...] = dinv

    return pl.pallas_call(
        body,
        grid=(GRID,),
        in_specs=[
            pl.BlockSpec((BR, IN_DIM), lambda i: (i, 0)),
            pl.BlockSpec((BR, 1), lambda i: (i, 0)),
            pl.BlockSpec((BR, 1), lambda i: (i, 0)),
            pl.BlockSpec((IN_DIM, H), lambda i: (0, 0)),
            pl.BlockSpec((1, H), lambda i: (0, 0)),
            pl.BlockSpec((H, H), lambda i: (0, 0)),
        ],
        out_specs=[
            pl.BlockSpec((BR, H), lambda i: (i, 0)),
            pl.BlockSpec((BR, 1), lambda i: (i, 0)),
        ],
        out_shape=[
            jax.ShapeDtypeStruct((NP, H), jnp.float32),
            jax.ShapeDtypeStruct((NP, 1), jnp.float32),
        ],
    )(xp, deg0, deg1, W0, b0, Wc0)


def _mid_call(acc0, acc1, hp, dinv, g, be, bc, Wc):
    def body(a0, a1, hp_ref, dv, g_ref, be_ref, bc_ref, wc_ref, out_ref):
        alpha = g_ref[...] * _INV_BN
        beta = bc_ref[...] * alpha + be_ref[...]
        dinv = dv[...]
        h = (a0[...] + a1[...] + hp_ref[...]) * dinv * alpha + beta
        h = jnp.maximum(h, 0.0)
        out_ref[...] = jnp.dot(
            h, wc_ref[...], preferred_element_type=jnp.float32) * dinv

    return pl.pallas_call(
        body,
        grid=(GRID,),
        in_specs=[
            pl.BlockSpec((BR, H), lambda i: (i, 0)),
            pl.BlockSpec((BR, H), lambda i: (i, 0)),
            pl.BlockSpec((BR, H), lambda i: (i, 0)),
            pl.BlockSpec((BR, 1), lambda i: (i, 0)),
            pl.BlockSpec((1, H), lambda i: (0, 0)),
            pl.BlockSpec((1, H), lambda i: (0, 0)),
            pl.BlockSpec((1, H), lambda i: (0, 0)),
            pl.BlockSpec((H, H), lambda i: (0, 0)),
        ],
        out_specs=pl.BlockSpec((BR, H), lambda i: (i, 0)),
        out_shape=jax.ShapeDtypeStruct((NP, H), jnp.float32),
    )(acc0, acc1, hp, dinv, g, be, bc, Wc)


def _final_call(acc0, acc1, hp, dinv, g, be, bc, batchp, W1, b1, W2, b2):
    def body(a0, a1, hp_ref, dv, g_ref, be_ref, bc_ref, b_ref,
             w1_ref, b1_ref, w2_ref, b2_ref, out_ref, sums, cnts):
        i = pl.program_id(0)

        @pl.when(i == 0)
        def _init():
            sums[...] = jnp.zeros_like(sums)
            cnts[...] = jnp.zeros_like(cnts)

        alpha = g_ref[...] * _INV_BN
        beta = bc_ref[...] * alpha + be_ref[...]
        dinv = dv[...]
        h = (a0[...] + a1[...] + hp_ref[...]) * dinv * alpha + beta
        h = jnp.maximum(h, 0.0)
        ids = lax.broadcasted_iota(jnp.int32, (BR, NG), 1)
        oh = (ids == b_ref[...]).astype(jnp.float32)
        sums[...] += lax.dot_general(
            oh, h, (((0,), (0,)), ((), ())), preferred_element_type=jnp.float32)
        cnts[...] += lax.dot_general(
            oh, jnp.ones((BR, 1), jnp.float32), (((0,), (0,)), ((), ())),
            preferred_element_type=jnp.float32)

        @pl.when(i == GRID - 1)
        def _fin():
            pooled = sums[...] / jnp.maximum(cnts[...], 1.0)
            z = jnp.dot(pooled, w1_ref[...], preferred_element_type=jnp.float32)
            z = jnp.maximum(z + b1_ref[...], 0.0)
            out_ref[...] = jnp.dot(
                z, w2_ref[...], preferred_element_type=jnp.float32) + b2_ref[...]

    return pl.pallas_call(
        body,
        grid=(GRID,),
        in_specs=[
            pl.BlockSpec((BR, H), lambda i: (i, 0)),
            pl.BlockSpec((BR, H), lambda i: (i, 0)),
            pl.BlockSpec((BR, H), lambda i: (i, 0)),
            pl.BlockSpec((BR, 1), lambda i: (i, 0)),
            pl.BlockSpec((1, H), lambda i: (0, 0)),
            pl.BlockSpec((1, H), lambda i: (0, 0)),
            pl.BlockSpec((1, H), lambda i: (0, 0)),
            pl.BlockSpec((BR, 1), lambda i: (i, 0)),
            pl.BlockSpec((H, H // 2), lambda i: (0, 0)),
            pl.BlockSpec((1, H // 2), lambda i: (0, 0)),
            pl.BlockSpec((H // 2, NCLS), lambda i: (0, 0)),
            pl.BlockSpec((1, NCLS), lambda i: (0, 0)),
        ],
        out_specs=pl.BlockSpec((NG, NCLS), lambda i: (0, 0)),
        out_shape=jax.ShapeDtypeStruct((NG, NCLS), jnp.float32),
        scratch_shapes=[
            pltpu.VMEM((NG, H), jnp.float32),
            pltpu.VMEM((NG, 1), jnp.float32),
        ],
    )(acc0, acc1, hp, dinv, g, be, bc, batchp, W1, b1, W2, b2)


def kernel(x, edge_index, batch, W0, b0, Wc0, bc0, Wc1, bc1, Wc2, bc2,
           g0, be0, g1, be1, g2, be2, W1, b1, W2, b2):
    f32 = jnp.float32
    xp = jnp.zeros((NP, IN_DIM), f32).at[:N].set(x)
    src = edge_index[0]
    dst = edge_index[1]
    pad = (N + (jnp.arange(EP - E, dtype=jnp.int32) % (NP - N))).astype(jnp.int32)
    padc = jnp.full((NBUF, NW, 1, CL), N, jnp.int32)
    srcp = jnp.concatenate([src, pad]).reshape(NW, NT, NBUF, CL).transpose(2, 0, 1, 3)
    srcp = jnp.concatenate([srcp, padc], axis=2)
    dstp = jnp.concatenate([dst, pad]).reshape(NW, NT, NBUF, CL).transpose(2, 0, 1, 3)
    dstp = jnp.concatenate([dstp, padc], axis=2)
    batchp = jnp.concatenate(
        [batch, jnp.full((NP - N,), NG, jnp.int32)]).reshape(NP, 1)

    ones_h = jnp.ones((CL, DEGW), f32)
    zerosd_h = jnp.zeros((RP, DEGW), f32)
    zeros_h = jnp.zeros((RP, H), f32)

    deg2 = _deg_call(dstp, ones_h, zerosd_h)
    deg0 = deg2[0, :, 0:1]
    deg1 = deg2[1, :, 0:1]

    hp0, dinv = _lead_call(xp, deg0, deg1, W0, b0.reshape(1, H), Wc0)
    accA = _scatter_call(hp0, srcp, dstp, zeros_h)
    hp1 = _mid_call(accA[0], accA[1], hp0, dinv,
                    g0.reshape(1, H), be0.reshape(1, H), bc0.reshape(1, H), Wc1)
    accB = _scatter_call(hp1, srcp, dstp, zeros_h)
    hp2 = _mid_call(accB[0], accB[1], hp1, dinv,
                    g1.reshape(1, H), be1.reshape(1, H), bc1.reshape(1, H), Wc2)
    accC = _scatter_call(hp2, srcp, dstp, zeros_h)
    out = _final_call(accC[0], accC[1], hp2, dinv,
                      g2.reshape(1, H), be2.reshape(1, H), bc2.reshape(1, H),
                      batchp, W1, b1.reshape(1, H // 2), W2, b2.reshape(1, NCLS))
    return out


# speedup vs baseline: 2.9597x; 2.9597x over previous
import functools
import math

import jax
import jax.numpy as jnp
from jax import lax
from jax.experimental import pallas as pl
from jax.experimental.pallas import tpu as pltpu
from jax.experimental.pallas import tpu_sc as plsc

N = 10000
IN_DIM = 128
H = 64
NCLS = 6
NG = 16
EPS = 1e-5

SC_CORES = 2
SC_TILES = 16
NW = SC_CORES * SC_TILES
CL = 128
E = 320000
NBUF = 8
NCH = -(-(-(-(E // NW) // CL)) // NBUF) * NBUF
NT = NCH // NBUF
EP = NW * NCH * CL
NP = 10240
RP = NP // SC_TILES
DEGW = 16

BR = 1024
GRID = NP // BR

_INV_BN = 1.0 / math.sqrt(1.0 + EPS)


def _sc_mesh():
    return plsc.VectorSubcoreMesh(
        core_axis_name="c", subcore_axis_name="s",
        num_cores=SC_CORES, num_subcores=SC_TILES)


def _deg_call(dstp, ones_h, zeros_h):
    @functools.partial(
        pl.kernel,
        out_type=jax.ShapeDtypeStruct((SC_CORES, NP, DEGW), jnp.float32),
        mesh=_sc_mesh(),
        scratch_types=[
            pltpu.VMEM((NCH, CL), jnp.int32),
            pltpu.VMEM((CL, DEGW), jnp.float32),
            pltpu.VMEM_SHARED((NP, DEGW), jnp.float32),
        ],
        compiler_params=pltpu.CompilerParams(use_tc_tiling_on_sc=False),
    )
    def k(dst_hbm, ones_hbm, zeros_hbm, out_hbm, didx, ones_v, acc):
        c = lax.axis_index("c")
        s = lax.axis_index("s")
        wid = c * SC_TILES + s
        for b in range(NBUF):
            pltpu.sync_copy(dst_hbm.at[b, wid], didx.at[pl.ds(b * NT, NT)])
        pltpu.sync_copy(ones_hbm, ones_v)
        pltpu.sync_copy(zeros_hbm, acc.at[pl.ds(s * RP, RP)])
        plsc.subcore_barrier()

        def step(j, carry):
            pltpu.sync_copy(ones_v, acc.at[didx.at[j]], add=True)
            return carry
        lax.fori_loop(0, NCH, step, 0)
        plsc.subcore_barrier()
        pltpu.sync_copy(acc.at[pl.ds(s * RP, RP)],
                        out_hbm.at[c, pl.ds(s * RP, RP)])

    return k(dstp, ones_h, zeros_h)


def _scatter_call(hp, srcp, dstp, zeros_h):
    @functools.partial(
        pl.kernel,
        out_type=jax.ShapeDtypeStruct((SC_CORES, NP, H), jnp.float32),
        mesh=_sc_mesh(),
        scratch_types=(
            [pltpu.VMEM((NT, CL), jnp.int32) for _ in range(2 * NBUF)]
            + [pltpu.VMEM((CL, H), jnp.float32) for _ in range(NBUF)]
            + [pltpu.VMEM_SHARED((NP, H), jnp.float32),
               pltpu.SemaphoreType.DMA]
        ),
        compiler_params=pltpu.CompilerParams(use_tc_tiling_on_sc=False),
    )
    def k(hp_hbm, src_hbm, dst_hbm, zeros_hbm, out_hbm, *rest):
        sidx = rest[:NBUF]
        didx = rest[NBUF:2 * NBUF]
        rows = rest[2 * NBUF:3 * NBUF]
        acc = rest[3 * NBUF]
        sem = rest[3 * NBUF + 1]
        c = lax.axis_index("c")
        s = lax.axis_index("s")
        wid = c * SC_TILES + s
        for b in range(NBUF):
            pltpu.sync_copy(src_hbm.at[b, wid], sidx[b])
            pltpu.sync_copy(dst_hbm.at[b, wid], didx[b])
        pltpu.sync_copy(zeros_hbm, acc.at[pl.ds(s * RP, RP)])
        plsc.subcore_barrier()

        def outer(t, carry):
            hs = [pltpu.async_copy(
                hp_hbm.at[sidx[b].at[t]], rows[b], sem)
                for b in range(NBUF)]
            for b in range(NBUF):
                hs[b].wait()
            for b in range(NBUF):
                pltpu.sync_copy(rows[b], acc.at[didx[b].at[t]], add=True)
            return carry
        lax.fori_loop(0, NT, outer, 0)
        plsc.subcore_barrier()
        pltpu.sync_copy(acc.at[pl.ds(s * RP, RP)],
                        out_hbm.at[c, pl.ds(s * RP, RP)])

    return k(hp, srcp, dstp, zeros_h)


def _lead_call(xp, deg0, deg1, W0, b0, Wc0):
    def body(x_ref, d0_ref, d1_ref, w0_ref, b0_ref, wc_ref, hp_ref, dinv_ref):
        dinv = lax.rsqrt(d0_ref[...] + d1_ref[...] + 1.0)
        h = jnp.dot(x_ref[...], w0_ref[...], preferred_element_type=jnp.float32)
        h = jnp.maximum(h + b0_ref[...], 0.0)
        hp = jnp.dot(h, wc_ref[...], preferred_element_type=jnp.float32)
        hp_ref[...] = hp * dinv
        dinv_ref[...] = dinv

    return pl.pallas_call(
        body,
        grid=(GRID,),
        in_specs=[
            pl.BlockSpec((BR, IN_DIM), lambda i: (i, 0)),
            pl.BlockSpec((BR, 1), lambda i: (i, 0)),
            pl.BlockSpec((BR, 1), lambda i: (i, 0)),
            pl.BlockSpec((IN_DIM, H), lambda i: (0, 0)),
            pl.BlockSpec((1, H), lambda i: (0, 0)),
            pl.BlockSpec((H, H), lambda i: (0, 0)),
        ],
        out_specs=[
            pl.BlockSpec((BR, H), lambda i: (i, 0)),
            pl.BlockSpec((BR, 1), lambda i: (i, 0)),
        ],
        out_shape=[
            jax.ShapeDtypeStruct((NP, H), jnp.float32),
            jax.ShapeDtypeStruct((NP, 1), jnp.float32),
        ],
    )(xp, deg0, deg1, W0, b0, Wc0)


def _mid_call(acc0, acc1, hp, dinv, g, be, bc, Wc):
    def body(a0, a1, hp_ref, dv, g_ref, be_ref, bc_ref, wc_ref, out_ref):
        alpha = g_ref[...] * _INV_BN
        beta = bc_ref[...] * alpha + be_ref[...]
        dinv = dv[...]
        h = (a0[...] + a1[...] + hp_ref[...]) * dinv * alpha + beta
        h = jnp.maximum(h, 0.0)
        out_ref[...] = jnp.dot(
            h, wc_ref[...], preferred_element_type=jnp.float32) * dinv

    return pl.pallas_call(
        body,
        grid=(GRID,),
        in_specs=[
            pl.BlockSpec((BR, H), lambda i: (i, 0)),
            pl.BlockSpec((BR, H), lambda i: (i, 0)),
            pl.BlockSpec((BR, H), lambda i: (i, 0)),
            pl.BlockSpec((BR, 1), lambda i: (i, 0)),
            pl.BlockSpec((1, H), lambda i: (0, 0)),
            pl.BlockSpec((1, H), lambda i: (0, 0)),
            pl.BlockSpec((1, H), lambda i: (0, 0)),
            pl.BlockSpec((H, H), lambda i: (0, 0)),
        ],
        out_specs=pl.BlockSpec((BR, H), lambda i: (i, 0)),
        out_shape=jax.ShapeDtypeStruct((NP, H), jnp.float32),
    )(acc0, acc1, hp, dinv, g, be, bc, Wc)


def _final_call(acc0, acc1, hp, dinv, g, be, bc, batchp, W1, b1, W2, b2):
    def body(a0, a1, hp_ref, dv, g_ref, be_ref, bc_ref, b_ref,
             w1_ref, b1_ref, w2_ref, b2_ref, out_ref, sums, cnts):
        i = pl.program_id(0)

        @pl.when(i == 0)
        def _init():
            sums[...] = jnp.zeros_like(sums)
            cnts[...] = jnp.zeros_like(cnts)

        alpha = g_ref[...] * _INV_BN
        beta = bc_ref[...] * alpha + be_ref[...]
        dinv = dv[...]
        h = (a0[...] + a1[...] + hp_ref[...]) * dinv * alpha + beta
        h = jnp.maximum(h, 0.0)
        ids = lax.broadcasted_iota(jnp.int32, (BR, NG), 1)
        oh = (ids == b_ref[...]).astype(jnp.float32)
        sums[...] += lax.dot_general(
            oh, h, (((0,), (0,)), ((), ())), preferred_element_type=jnp.float32)
        cnts[...] += lax.dot_general(
            oh, jnp.ones((BR, 1), jnp.float32), (((0,), (0,)), ((), ())),
            preferred_element_type=jnp.float32)

        @pl.when(i == GRID - 1)
        def _fin():
            pooled = sums[...] / jnp.maximum(cnts[...], 1.0)
            z = jnp.dot(pooled, w1_ref[...], preferred_element_type=jnp.float32)
            z = jnp.maximum(z + b1_ref[...], 0.0)
            out_ref[...] = jnp.dot(
                z, w2_ref[...], preferred_element_type=jnp.float32) + b2_ref[...]

    return pl.pallas_call(
        body,
        grid=(GRID,),
        in_specs=[
            pl.BlockSpec((BR, H), lambda i: (i, 0)),
            pl.BlockSpec((BR, H), lambda i: (i, 0)),
            pl.BlockSpec((BR, H), lambda i: (i, 0)),
            pl.BlockSpec((BR, 1), lambda i: (i, 0)),
            pl.BlockSpec((1, H), lambda i: (0, 0)),
            pl.BlockSpec((1, H), lambda i: (0, 0)),
            pl.BlockSpec((1, H), lambda i: (0, 0)),
            pl.BlockSpec((BR, 1), lambda i: (i, 0)),
            pl.BlockSpec((H, H // 2), lambda i: (0, 0)),
            pl.BlockSpec((1, H // 2), lambda i: (0, 0)),
            pl.BlockSpec((H // 2, NCLS), lambda i: (0, 0)),
            pl.BlockSpec((1, NCLS), lambda i: (0, 0)),
        ],
        out_specs=pl.BlockSpec((NG, NCLS), lambda i: (0, 0)),
        out_shape=jax.ShapeDtypeStruct((NG, NCLS), jnp.float32),
        scratch_shapes=[
            pltpu.VMEM((NG, H), jnp.float32),
            pltpu.VMEM((NG, 1), jnp.float32),
        ],
    )(acc0, acc1, hp, dinv, g, be, bc, batchp, W1, b1, W2, b2)


def kernel(x, edge_index, batch, W0, b0, Wc0, bc0, Wc1, bc1, Wc2, bc2,
           g0, be0, g1, be1, g2, be2, W1, b1, W2, b2):
    f32 = jnp.float32
    xp = jnp.zeros((NP, IN_DIM), f32).at[:N].set(x)
    src = edge_index[0]
    dst = edge_index[1]
    pad = (N + (jnp.arange(EP - E, dtype=jnp.int32) % (NP - N))).astype(jnp.int32)
    srcp = jnp.concatenate([src, pad]).reshape(NW, NT, NBUF, CL).transpose(2, 0, 1, 3)
    dstp = jnp.concatenate([dst, pad]).reshape(NW, NT, NBUF, CL).transpose(2, 0, 1, 3)
    batchp = jnp.concatenate(
        [batch, jnp.full((NP - N,), NG, jnp.int32)]).reshape(NP, 1)

    ones_h = jnp.ones((CL, DEGW), f32)
    zerosd_h = jnp.zeros((RP, DEGW), f32)
    zeros_h = jnp.zeros((RP, H), f32)

    deg2 = _deg_call(dstp, ones_h, zerosd_h)
    deg0 = deg2[0, :, 0:1]
    deg1 = deg2[1, :, 0:1]

    hp0, dinv = _lead_call(xp, deg0, deg1, W0, b0.reshape(1, H), Wc0)
    accA = _scatter_call(hp0, srcp, dstp, zeros_h)
    hp1 = _mid_call(accA[0], accA[1], hp0, dinv,
                    g0.reshape(1, H), be0.reshape(1, H), bc0.reshape(1, H), Wc1)
    accB = _scatter_call(hp1, srcp, dstp, zeros_h)
    hp2 = _mid_call(accB[0], accB[1], hp1, dinv,
                    g1.reshape(1, H), be1.reshape(1, H), bc1.reshape(1, H), Wc2)
    accC = _scatter_call(hp2, srcp, dstp, zeros_h)
    out = _final_call(accC[0], accC[1], hp2, dinv,
                      g2.reshape(1, H), be2.reshape(1, H), bc2.reshape(1, H),
                      batchp, W1, b1.reshape(1, H // 2), W2, b2.reshape(1, NCLS))
    return out
